# Initial kernel scaffold; baseline (speedup 1.0000x reference)
import jax
import jax.numpy as jnp
from jax.experimental import pallas as pl


def kernel(input, target, mask):
    raise NotImplementedError("write your pallas kernel here")


# speedup vs baseline: 5.0319x; 5.0319x over previous
import functools
import math

import jax
import jax.numpy as jnp
from jax import lax
from jax.experimental import pallas as pl
from jax.experimental.pallas import tpu as pltpu
from jax.experimental.pallas import tpu_sc as plsc

_SMOOTHING = 0.1
_CONFIDENCE = 1.0 - _SMOOTHING

_ROW_BLOCK = 256


def _tc_rowsum_body(x_ref, m_ref, out_ref):
    i = pl.program_id(0)

    @pl.when(i == 0)
    def _init():
        out_ref[0] = 0.0
        out_ref[1] = 0.0

    rs = jnp.sum(x_ref[...], axis=1)
    m = m_ref[0, 0, :]
    out_ref[0] += jnp.sum(rs * m)
    out_ref[1] += jnp.sum(m)


def _tc_masked_rowsum(x, m):
    n, v = x.shape
    grid = n // _ROW_BLOCK
    m3 = m.reshape(grid, 1, _ROW_BLOCK)
    return pl.pallas_call(
        _tc_rowsum_body,
        grid=(grid,),
        in_specs=[
            pl.BlockSpec((_ROW_BLOCK, v), lambda i: (i, 0)),
            pl.BlockSpec((1, 1, _ROW_BLOCK), lambda i: (i, 0, 0)),
        ],
        out_specs=pl.BlockSpec(memory_space=pltpu.SMEM),
        out_shape=jax.ShapeDtypeStruct((2,), jnp.float32),
    )(x, m3)


def _sc_masked_gather_partials(x_flat, flat_idx, m):
    n = flat_idx.shape[0]
    info = plsc.get_sparse_core_info()
    num_cores, num_subcores, num_lanes = (
        info.num_cores, info.num_subcores, info.num_lanes)
    nw = num_cores * num_subcores
    bpw = n // nw
    chunk = 128
    nchunks = bpw // chunk
    mesh = plsc.VectorSubcoreMesh(core_axis_name="c", subcore_axis_name="s")

    @functools.partial(
        pl.kernel,
        mesh=mesh,
        out_type=jax.ShapeDtypeStruct((nw, num_lanes), jnp.float32),
        scratch_types=[
            pltpu.VMEM((chunk,), jnp.int32),
            pltpu.VMEM((chunk,), jnp.float32),
            pltpu.VMEM((chunk,), jnp.float32),
            pltpu.VMEM((num_lanes,), jnp.float32),
            pltpu.SemaphoreType.DMA,
        ],
    )
    def k(x_hbm, idx_hbm, m_hbm, out_hbm, idx_v, vals_v, m_v, acc_v, sem):
        wid = lax.axis_index("s") * num_cores + lax.axis_index("c")
        base = wid * bpw
        acc = jnp.zeros((num_lanes,), jnp.float32)
        for c in range(nchunks):
            off = base + c * chunk
            pltpu.sync_copy(idx_hbm.at[pl.ds(off, chunk)], idx_v)
            pltpu.sync_copy(m_hbm.at[pl.ds(off, chunk)], m_v)
            pltpu.async_copy(x_hbm.at[idx_v], vals_v, sem).wait()
            for j in range(chunk // num_lanes):
                sl = pl.ds(j * num_lanes, num_lanes)
                acc = acc + vals_v[sl] * m_v[sl]
        acc_v[...] = acc
        pltpu.sync_copy(acc_v, out_hbm.at[wid])

    return k(x_flat, flat_idx, m)


def kernel(input, target, mask):
    b, t, v = input.shape
    n = b * t
    x = input.reshape(n, v)
    m = mask.reshape(n).astype(jnp.float32)
    tgt = target.reshape(n).astype(jnp.int32)
    flat_idx = jnp.arange(n, dtype=jnp.int32) * v + tgt

    tc_out = _tc_masked_rowsum(x, m)
    sc_partials = _sc_masked_gather_partials(x.reshape(-1), flat_idx, m)

    a = tc_out[0]
    msum = tc_out[1]
    tdot = jnp.sum(sc_partials)

    eps = _SMOOTHING / (v - 1)
    const = (v - 1) * eps * math.log(eps) + _CONFIDENCE * math.log(_CONFIDENCE)
    loss = (const * msum - eps * a - (_CONFIDENCE - eps) * tdot) / msum
    return loss


# speedup vs baseline: 14.0421x; 2.7906x over previous
import functools
import math

import jax
import jax.numpy as jnp
from jax import lax
from jax.experimental import pallas as pl
from jax.experimental.pallas import tpu as pltpu
from jax.experimental.pallas import tpu_sc as plsc

_SMOOTHING = 0.1
_CONFIDENCE = 1.0 - _SMOOTHING

_ROW_BLOCK = 256


def _tc_rowsum_body(x_ref, m_ref, out_ref):
    i = pl.program_id(0)

    @pl.when(i == 0)
    def _init():
        out_ref[0] = 0.0
        out_ref[1] = 0.0

    rs = jnp.sum(x_ref[...], axis=1)
    m = m_ref[0, 0, :]
    out_ref[0] += jnp.sum(rs * m)
    out_ref[1] += jnp.sum(m)


def _tc_masked_rowsum(x, m):
    n, v = x.shape
    grid = n // _ROW_BLOCK
    m3 = m.reshape(grid, 1, _ROW_BLOCK)
    return pl.pallas_call(
        _tc_rowsum_body,
        grid=(grid,),
        in_specs=[
            pl.BlockSpec((_ROW_BLOCK, v), lambda i: (i, 0)),
            pl.BlockSpec((1, 1, _ROW_BLOCK), lambda i: (i, 0, 0)),
        ],
        out_specs=pl.BlockSpec(memory_space=pltpu.SMEM),
        out_shape=jax.ShapeDtypeStruct((2,), jnp.float32),
    )(x, m3)


def _sc_masked_gather_partials(x_flat, flat_idx, m):
    n = flat_idx.shape[0]
    info = plsc.get_sparse_core_info()
    num_cores, num_subcores, num_lanes = (
        info.num_cores, info.num_subcores, info.num_lanes)
    nw = num_cores * num_subcores
    bpw = n // nw
    chunk = 128
    nchunks = bpw // chunk
    mesh = plsc.VectorSubcoreMesh(core_axis_name="c", subcore_axis_name="s")

    @functools.partial(
        pl.kernel,
        mesh=mesh,
        out_type=jax.ShapeDtypeStruct((nw, num_lanes), jnp.float32),
        scratch_types=[
            pltpu.VMEM((chunk,), jnp.int32),
            pltpu.VMEM((chunk,), jnp.float32),
            pltpu.VMEM((chunk,), jnp.float32),
            pltpu.VMEM((num_lanes,), jnp.float32),
            pltpu.SemaphoreType.DMA,
        ],
    )
    def k(x_hbm, idx_hbm, m_hbm, out_hbm, idx_v, vals_v, m_v, acc_v, sem):
        wid = lax.axis_index("s") * num_cores + lax.axis_index("c")
        base = wid * bpw
        acc = jnp.zeros((num_lanes,), jnp.float32)
        for c in range(nchunks):
            off = base + c * chunk
            pltpu.sync_copy(idx_hbm.at[pl.ds(off, chunk)], idx_v)
            pltpu.sync_copy(m_hbm.at[pl.ds(off, chunk)], m_v)
            pltpu.async_copy(x_hbm.at[idx_v], vals_v, sem).wait()
            for j in range(chunk // num_lanes):
                sl = pl.ds(j * num_lanes, num_lanes)
                acc = acc + vals_v[sl] * m_v[sl]
        acc_v[...] = acc
        pltpu.sync_copy(acc_v, out_hbm.at[wid])

    return k(x_flat, flat_idx, m)


def kernel(input, target, mask):
    b, t, v = input.shape
    n = b * t
    x = input.reshape(n, v)
    m = mask.reshape(n).astype(jnp.float32)
    tgt = target.reshape(n).astype(jnp.int32)

    xt = input.reshape(n // 8, 8, v // 128, 128)
    xt = xt.transpose(0, 2, 1, 3).reshape(-1)
    r = jnp.arange(n, dtype=jnp.int32)
    flat_idx = ((r // 8) * (8 * v) + (tgt // 128) * 1024
                + (r % 8) * 128 + (tgt % 128))

    tc_out = _tc_masked_rowsum(x, m)
    sc_partials = _sc_masked_gather_partials(xt, flat_idx, m)

    a = tc_out[0]
    msum = tc_out[1]
    tdot = jnp.sum(sc_partials)

    eps = _SMOOTHING / (v - 1)
    const = (v - 1) * eps * math.log(eps) + _CONFIDENCE * math.log(_CONFIDENCE)
    loss = (const * msum - eps * a - (_CONFIDENCE - eps) * tdot) / msum
    return loss
